# Initial kernel scaffold; baseline (speedup 1.0000x reference)
import functools

import jax
import jax.numpy as jnp
from jax import lax
from jax.experimental import pallas as pl
from jax.experimental.pallas import tpu as pltpu
from jax.experimental.pallas import tpu_sc as plsc

N = 10000
NP = 10240
D = 128
F = 64
NC = 2
NS = 16
NT = NC * NS
K = 128
E = 320000
E_TOT = E + NP
CH = -(-E_TOT // (NT * K))
E_PAD = NT * CH * K
RPT = NP // NS

_mesh = plsc.VectorSubcoreMesh(core_axis_name="c", subcore_axis_name="s")


@functools.partial(
    pl.kernel,
    out_type=jax.ShapeDtypeStruct((NC, NP), jnp.float32),
    mesh=_mesh,
    scratch_types=[
        pltpu.VMEM((CH, K), jnp.int32),
        pltpu.VMEM((CH, K), jnp.float32),
        pltpu.VMEM_SHARED((NP,), jnp.float32),
    ],
)
def _deg_kernel(col_hbm, w_hbm, z1_hbm, deg_out, col_v, w_v, deg_sh):
    c = lax.axis_index("c")
    s = lax.axis_index("s")
    wid = s * NC + c
    pltpu.sync_copy(col_hbm.at[wid], col_v)
    pltpu.sync_copy(w_hbm.at[wid], w_v)
    pltpu.sync_copy(z1_hbm, deg_sh.at[pl.ds(s * RPT, RPT)])
    plsc.subcore_barrier()

    def body(i, carry):
        pltpu.sync_copy(w_v.at[i], deg_sh.at[col_v.at[i]], add=True)
        return carry

    lax.fori_loop(0, CH, body, 0)
    plsc.subcore_barrier()
    pltpu.sync_copy(deg_sh.at[pl.ds(s * RPT, RPT)],
                    deg_out.at[c, pl.ds(s * RPT, RPT)])


@functools.partial(
    pl.kernel,
    out_type=jax.ShapeDtypeStruct((NC, NP, F), jnp.float32),
    mesh=_mesh,
    scratch_types=[
        pltpu.VMEM((CH, K), jnp.int32),
        pltpu.VMEM((CH, K), jnp.int32),
        pltpu.VMEM((CH, K), jnp.float32),
        pltpu.VMEM((NP,), jnp.float32),
        pltpu.VMEM((K, F), jnp.float32),
        pltpu.VMEM((K,), jnp.float32),
        pltpu.VMEM_SHARED((NP, F), jnp.float32),
        pltpu.SemaphoreType.DMA,
    ],
)
def _edge_kernel(xw_hbm, row_hbm, col_hbm, w_hbm, dis_hbm, z64_hbm, acc_out,
                 row_v, col_v, w_v, dis_v, gbuf, nbuf, acc_sh, sem):
    c = lax.axis_index("c")
    s = lax.axis_index("s")
    wid = s * NC + c
    pltpu.sync_copy(row_hbm.at[wid], row_v)
    pltpu.sync_copy(col_hbm.at[wid], col_v)
    pltpu.sync_copy(w_hbm.at[wid], w_v)
    pltpu.sync_copy(dis_hbm, dis_v)
    pltpu.sync_copy(z64_hbm, acc_sh.at[pl.ds(s * RPT, RPT)])
    plsc.subcore_barrier()

    def chunk(i, carry):
        for j in range(K // 16):
            r16 = row_v[i, pl.ds(j * 16, 16)]
            c16 = col_v[i, pl.ds(j * 16, 16)]
            w16 = w_v[i, pl.ds(j * 16, 16)]
            n16 = plsc.load_gather(dis_v, [r16]) * w16 \
                * plsc.load_gather(dis_v, [c16])
            nbuf[pl.ds(j * 16, 16)] = n16
        pltpu.async_copy(xw_hbm.at[row_v.at[i]], gbuf, sem).wait()

        def scale(k8, carry2):
            for u in range(8):
                k = k8 * 8 + u
                sc = nbuf[k]
                for j in range(F // 16):
                    gbuf[k, pl.ds(j * 16, 16)] = gbuf[k, pl.ds(j * 16, 16)] * sc
            return carry2

        lax.fori_loop(0, K // 8, scale, 0)
        pltpu.sync_copy(gbuf, acc_sh.at[col_v.at[i]], add=True)
        return carry

    lax.fori_loop(0, CH, chunk, 0)
    plsc.subcore_barrier()
    pltpu.sync_copy(acc_sh.at[pl.ds(s * RPT, RPT)],
                    acc_out.at[c, pl.ds(s * RPT, RPT)])


def _dis_body(degp_ref, dis_ref):
    d = degp_ref[0] + degp_ref[1]
    dis_ref[...] = jax.lax.rsqrt(d)


def _dis_kernel(degp):
    return pl.pallas_call(
        _dis_body,
        out_shape=jax.ShapeDtypeStruct((NP // 128, 128), jnp.float32),
    )(degp.reshape(NC, NP // 128, 128))


def _mm_body(x_ref, w_ref, o_ref):
    o_ref[...] = jnp.dot(x_ref[...], w_ref[...],
                         preferred_element_type=jnp.float32)


def _mm(x, w):
    br = 1024
    din, dout = w.shape
    return pl.pallas_call(
        _mm_body,
        grid=(NP // br,),
        in_specs=[
            pl.BlockSpec((br, din), lambda i: (i, 0)),
            pl.BlockSpec((din, dout), lambda i: (0, 0)),
        ],
        out_specs=pl.BlockSpec((br, dout), lambda i: (i, 0)),
        out_shape=jax.ShapeDtypeStruct((NP, dout), jnp.float32),
    )(x, w)


def _mid_body(acc_ref, b_ref, w2_ref, o_ref):
    h = jax.nn.relu(acc_ref[0] + acc_ref[1] + b_ref[...])
    o_ref[...] = jnp.dot(h, w2_ref[...], preferred_element_type=jnp.float32)


def _mid(acc, b1, w2):
    br = 1024
    return pl.pallas_call(
        _mid_body,
        grid=(NP // br,),
        in_specs=[
            pl.BlockSpec((NC, br, F), lambda i: (0, i, 0)),
            pl.BlockSpec((1, F), lambda i: (0, 0)),
            pl.BlockSpec((F, F), lambda i: (0, 0)),
        ],
        out_specs=pl.BlockSpec((br, F), lambda i: (i, 0)),
        out_shape=jax.ShapeDtypeStruct((NP, F), jnp.float32),
    )(acc, b1.reshape(1, F), w2)


def _fin_body(acc_ref, b_ref, o_ref):
    o_ref[...] = acc_ref[0] + acc_ref[1] + b_ref[...]


def _fin(acc, b2):
    br = 1024
    return pl.pallas_call(
        _fin_body,
        grid=(NP // br,),
        in_specs=[
            pl.BlockSpec((NC, br, F), lambda i: (0, i, 0)),
            pl.BlockSpec((1, F), lambda i: (0, 0)),
        ],
        out_specs=pl.BlockSpec((br, F), lambda i: (i, 0)),
        out_shape=jax.ShapeDtypeStruct((NP, F), jnp.float32),
    )(acc, b2.reshape(1, F))


def kernel(data, edge_idx, edge_weight, W1, b1, W2, b2):
    pad = E_PAD - E_TOT
    loop = jnp.arange(NP, dtype=jnp.int32)
    row = jnp.concatenate([edge_idx[0].astype(jnp.int32), loop,
                           jnp.zeros((pad,), jnp.int32)])
    col = jnp.concatenate([edge_idx[1].astype(jnp.int32), loop,
                           jnp.zeros((pad,), jnp.int32)])
    w = jnp.concatenate([edge_weight, jnp.ones((NP,), jnp.float32),
                         jnp.zeros((pad,), jnp.float32)])
    row3 = row.reshape(NT, CH, K)
    col3 = col.reshape(NT, CH, K)
    w3 = w.reshape(NT, CH, K)
    z1 = jnp.zeros((RPT,), jnp.float32)
    z64 = jnp.zeros((RPT, F), jnp.float32)
    data_p = jnp.pad(data, ((0, NP - N), (0, 0)))

    degp = _deg_kernel(col3, w3, z1)
    dis = _dis_kernel(degp).reshape(NP)

    xw1 = _mm(data_p, W1)
    acc1 = _edge_kernel(xw1, row3, col3, w3, dis, z64)
    xw2 = _mid(acc1, b1, W2)
    acc2 = _edge_kernel(xw2, row3, col3, w3, dis, z64)
    out = _fin(acc2, b2)
    return out[:N]


# speedup vs baseline: 19.0437x; 19.0437x over previous
import functools

import jax
import jax.numpy as jnp
from jax import lax
from jax.experimental import pallas as pl
from jax.experimental.pallas import tpu as pltpu
from jax.experimental.pallas import tpu_sc as plsc

N = 10000
NP = 10240
D = 128
F = 64
NC = 2
NS = 16
NT = NC * NS
K = 128
E = 320000
E_TOT = E + NP
CH = -(-E_TOT // (NT * K))
E_PAD = NT * CH * K
RPT = NP // NS

_mesh = plsc.VectorSubcoreMesh(core_axis_name="c", subcore_axis_name="s")


@functools.partial(
    pl.kernel,
    out_type=jax.ShapeDtypeStruct((NC, NP), jnp.float32),
    mesh=_mesh,
    compiler_params=pltpu.CompilerParams(needs_layout_passes=False, use_tc_tiling_on_sc=False),
    scratch_types=[
        pltpu.VMEM((CH, K), jnp.int32),
        pltpu.VMEM((CH, K), jnp.float32),
        pltpu.VMEM_SHARED((NP,), jnp.float32),
    ],
)
def _deg_kernel(col_hbm, w_hbm, z1_hbm, deg_out, col_v, w_v, deg_sh):
    c = lax.axis_index("c")
    s = lax.axis_index("s")
    wid = s * NC + c
    pltpu.sync_copy(col_hbm.at[wid], col_v)
    pltpu.sync_copy(w_hbm.at[wid], w_v)
    pltpu.sync_copy(z1_hbm, deg_sh.at[pl.ds(s * RPT, RPT)])
    plsc.subcore_barrier()

    def body(i, carry):
        pltpu.sync_copy(w_v.at[i], deg_sh.at[col_v.at[i]], add=True)
        return carry

    lax.fori_loop(0, CH, body, 0)
    plsc.subcore_barrier()
    pltpu.sync_copy(deg_sh.at[pl.ds(s * RPT, RPT)],
                    deg_out.at[c, pl.ds(s * RPT, RPT)])


@functools.partial(
    pl.kernel,
    out_type=jax.ShapeDtypeStruct((NC, NP, F), jnp.float32),
    mesh=_mesh,
    compiler_params=pltpu.CompilerParams(needs_layout_passes=False, use_tc_tiling_on_sc=False),
    scratch_types=[
        pltpu.VMEM((CH, K), jnp.int32),
        pltpu.VMEM((CH, K), jnp.int32),
        pltpu.VMEM((CH, K), jnp.float32),
        pltpu.VMEM((NP,), jnp.float32),
        pltpu.VMEM((K, F), jnp.float32),
        pltpu.VMEM((K,), jnp.float32),
        pltpu.VMEM_SHARED((NP, F), jnp.float32),
        pltpu.SemaphoreType.DMA,
    ],
)
def _edge_kernel(xw_hbm, row_hbm, col_hbm, w_hbm, dis_hbm, z64_hbm, acc_out,
                 row_v, col_v, w_v, dis_v, gbuf, nbuf, acc_sh, sem):
    c = lax.axis_index("c")
    s = lax.axis_index("s")
    wid = s * NC + c
    pltpu.sync_copy(row_hbm.at[wid], row_v)
    pltpu.sync_copy(col_hbm.at[wid], col_v)
    pltpu.sync_copy(w_hbm.at[wid], w_v)
    pltpu.sync_copy(dis_hbm, dis_v)
    pltpu.sync_copy(z64_hbm, acc_sh.at[pl.ds(s * RPT, RPT)])
    plsc.subcore_barrier()

    def chunk(i, carry):
        for j in range(K // 16):
            r16 = row_v[i, pl.ds(j * 16, 16)]
            c16 = col_v[i, pl.ds(j * 16, 16)]
            w16 = w_v[i, pl.ds(j * 16, 16)]
            n16 = plsc.load_gather(dis_v, [r16]) * w16 \
                * plsc.load_gather(dis_v, [c16])
            nbuf[pl.ds(j * 16, 16)] = n16
        pltpu.async_copy(xw_hbm.at[row_v.at[i]], gbuf, sem).wait()

        def scale(g, carry2):
            n16 = nbuf[pl.ds(g * 16, 16)]
            base = g * 16
            for u in range(16):
                sc = n16[u]
                for j in range(F // 16):
                    gbuf[base + u, pl.ds(j * 16, 16)] = (
                        gbuf[base + u, pl.ds(j * 16, 16)] * sc)
            return carry2

        lax.fori_loop(0, K // 16, scale, 0)
        pltpu.sync_copy(gbuf, acc_sh.at[col_v.at[i]], add=True)
        return carry

    lax.fori_loop(0, CH, chunk, 0)
    plsc.subcore_barrier()
    pltpu.sync_copy(acc_sh.at[pl.ds(s * RPT, RPT)],
                    acc_out.at[c, pl.ds(s * RPT, RPT)])


def _dis_body(degp_ref, dis_ref):
    d = degp_ref[0] + degp_ref[1]
    dis_ref[...] = jax.lax.rsqrt(d)


def _dis_kernel(degp):
    return pl.pallas_call(
        _dis_body,
        out_shape=jax.ShapeDtypeStruct((NP // 128, 128), jnp.float32),
    )(degp.reshape(NC, NP // 128, 128))


def _mm_body(x_ref, w_ref, o_ref):
    o_ref[...] = jnp.dot(x_ref[...], w_ref[...],
                         preferred_element_type=jnp.float32)


def _mm(x, w):
    br = 1024
    din, dout = w.shape
    return pl.pallas_call(
        _mm_body,
        grid=(NP // br,),
        in_specs=[
            pl.BlockSpec((br, din), lambda i: (i, 0)),
            pl.BlockSpec((din, dout), lambda i: (0, 0)),
        ],
        out_specs=pl.BlockSpec((br, dout), lambda i: (i, 0)),
        out_shape=jax.ShapeDtypeStruct((NP, dout), jnp.float32),
    )(x, w)


def _mid_body(acc_ref, b_ref, w2_ref, o_ref):
    h = jax.nn.relu(acc_ref[0] + acc_ref[1] + b_ref[...])
    o_ref[...] = jnp.dot(h, w2_ref[...], preferred_element_type=jnp.float32)


def _mid(acc, b1, w2):
    br = 1024
    return pl.pallas_call(
        _mid_body,
        grid=(NP // br,),
        in_specs=[
            pl.BlockSpec((NC, br, F), lambda i: (0, i, 0)),
            pl.BlockSpec((1, F), lambda i: (0, 0)),
            pl.BlockSpec((F, F), lambda i: (0, 0)),
        ],
        out_specs=pl.BlockSpec((br, F), lambda i: (i, 0)),
        out_shape=jax.ShapeDtypeStruct((NP, F), jnp.float32),
    )(acc, b1.reshape(1, F), w2)


def _fin_body(acc_ref, b_ref, o_ref):
    o_ref[...] = acc_ref[0] + acc_ref[1] + b_ref[...]


def _fin(acc, b2):
    br = 1024
    return pl.pallas_call(
        _fin_body,
        grid=(NP // br,),
        in_specs=[
            pl.BlockSpec((NC, br, F), lambda i: (0, i, 0)),
            pl.BlockSpec((1, F), lambda i: (0, 0)),
        ],
        out_specs=pl.BlockSpec((br, F), lambda i: (i, 0)),
        out_shape=jax.ShapeDtypeStruct((NP, F), jnp.float32),
    )(acc, b2.reshape(1, F))


def kernel(data, edge_idx, edge_weight, W1, b1, W2, b2):
    pad = E_PAD - E_TOT
    loop = jnp.arange(NP, dtype=jnp.int32)
    row = jnp.concatenate([edge_idx[0].astype(jnp.int32), loop,
                           jnp.zeros((pad,), jnp.int32)])
    col = jnp.concatenate([edge_idx[1].astype(jnp.int32), loop,
                           jnp.zeros((pad,), jnp.int32)])
    w = jnp.concatenate([edge_weight, jnp.ones((NP,), jnp.float32),
                         jnp.zeros((pad,), jnp.float32)])
    row3 = row.reshape(NT, CH, K)
    col3 = col.reshape(NT, CH, K)
    w3 = w.reshape(NT, CH, K)
    z1 = jnp.zeros((RPT,), jnp.float32)
    z64 = jnp.zeros((RPT, F), jnp.float32)
    data_p = jnp.pad(data, ((0, NP - N), (0, 0)))

    degp = _deg_kernel(col3, w3, z1)
    dis = _dis_kernel(degp).reshape(NP)

    xw1 = _mm(data_p, W1)
    acc1 = _edge_kernel(xw1, row3, col3, w3, dis, z64)
    xw2 = _mid(acc1, b1, W2)
    acc2 = _edge_kernel(xw2, row3, col3, w3, dis, z64)
    out = _fin(acc2, b2)
    return out[:N]
